# baseline (speedup 1.0000x reference)
import functools

import jax
import jax.numpy as jnp
from jax import lax
from jax.experimental import pallas as pl
from jax.experimental.pallas import tpu as pltpu
from jax.experimental.pallas import tpu_sc as plsc

_D = 64
_V = 1000000
_B = 4096 * 200

_info = plsc.get_sparse_core_info()
_NC, _NS = _info.num_cores, _info.num_subcores
_NW = _NC * _NS
_BPW = _B // _NW
_CHUNK = 160
_NCHUNK = _BPW // _CHUNK
_GW = _CHUNK // 2


def _sc_gather(idx_hbm, table_hbm, out_hbm,
               idxv0, idxv1, idx20, idx21, par0, par1,
               wide0, wide1, sel0, sel1,
               sem_idx, semg0, semg1, semo0, semo1):
    wid = lax.axis_index("s") * _NC + lax.axis_index("c")
    base = wid * _BPW
    idx_v = (idxv0, idxv1)
    idx2_v = (idx20, idx21)
    par_v = (par0, par1)
    wide_v = (wide0, wide1)
    sel_v = (sel0, sel1)
    sem_g = (semg0, semg1)
    sem_out = (semo0, semo1)

    def stage(i, s):
        off = pl.multiple_of(base + i * _CHUNK, _CHUNK)
        pltpu.make_async_copy(idx_hbm.at[pl.ds(off, _CHUNK)],
                              idx_v[s].at[pl.ds(0, _CHUNK)], sem_idx).wait()

        @pl.when(i + 1 < _NCHUNK)
        def _():
            offn = pl.multiple_of(base + (i + 1) * _CHUNK, _CHUNK)
            pltpu.async_copy(idx_hbm.at[pl.ds(offn, _CHUNK)],
                             idx_v[1 - s].at[pl.ds(0, _CHUNK)], sem_idx)

        for j in range(_CHUNK // 16):
            t = idx_v[s][pl.ds(j * 16, 16)]
            idx2_v[s][pl.ds(j * 16, 16)] = lax.shift_right_logical(t, 1)
            par_v[s][pl.ds(j * 16, 16)] = (t & 1) * 64
        for j in range(_CHUNK // _GW):
            pltpu.async_copy(
                table_hbm.at[idx2_v[s].at[pl.ds(j * _GW, _GW)]],
                wide_v[s].at[pl.ds(j * _GW, _GW)], sem_g[s])

    def complete(i, s, drain):
        off = pl.multiple_of(base + i * _CHUNK, _CHUNK)
        for j in range(_CHUNK // _GW):
            pltpu.make_async_copy(
                table_hbm.at[idx2_v[s].at[pl.ds(j * _GW, _GW)]],
                wide_v[s].at[pl.ds(j * _GW, _GW)], sem_g[s]).wait()
        if drain:
            @pl.when(i >= 2)
            def _():
                pltpu.make_async_copy(sel_v[s],
                                      out_hbm.at[pl.ds(off, _CHUNK)],
                                      sem_out[s]).wait()

        @plsc.parallel_loop(0, _CHUNK, unroll=8)
        def _(r):
            b = par_v[s][pl.ds(r, 16)][0]
            for j in range(4):
                sel_v[s][r, pl.ds(j * 16, 16)] = (
                    wide_v[s][r, pl.ds(b + j * 16, 16)])

        pltpu.async_copy(sel_v[s], out_hbm.at[pl.ds(off, _CHUNK)],
                         sem_out[s])

    pltpu.async_copy(idx_hbm.at[pl.ds(pl.multiple_of(base, _CHUNK), _CHUNK)],
                     idx_v[0].at[pl.ds(0, _CHUNK)], sem_idx)
    stage(0, 0)

    def pair(g, _):
        i = g * 2
        stage(i + 1, 1)
        complete(i, 0, True)

        @pl.when(i + 2 < _NCHUNK)
        def _():
            stage(i + 2, 0)
        complete(i + 1, 1, True)
        return ()

    lax.fori_loop(0, _NCHUNK // 2, pair, ())
    for s in range(2):
        off = pl.multiple_of(base + (_NCHUNK - 2 + s) * _CHUNK, _CHUNK)
        pltpu.make_async_copy(sel_v[s], out_hbm.at[pl.ds(off, _CHUNK)],
                              sem_out[s]).wait()


@jax.jit
def _embed(token_ids_flat, wpairs):
    mesh = plsc.VectorSubcoreMesh(core_axis_name="c", subcore_axis_name="s")
    k = functools.partial(
        pl.kernel,
        mesh=mesh,
        compiler_params=pltpu.CompilerParams(needs_layout_passes=False),
        out_type=jax.ShapeDtypeStruct((_B, _D), jnp.float32),
        scratch_types=[
            pltpu.VMEM((_CHUNK + 16,), jnp.int32),
            pltpu.VMEM((_CHUNK + 16,), jnp.int32),
            pltpu.VMEM((_CHUNK,), jnp.int32),
            pltpu.VMEM((_CHUNK,), jnp.int32),
            pltpu.VMEM((_CHUNK + 16,), jnp.int32),
            pltpu.VMEM((_CHUNK + 16,), jnp.int32),
            pltpu.VMEM((_CHUNK, 128), jnp.float32),
            pltpu.VMEM((_CHUNK, 128), jnp.float32),
            pltpu.VMEM((_CHUNK, _D), jnp.float32),
            pltpu.VMEM((_CHUNK, _D), jnp.float32),
            pltpu.SemaphoreType.DMA,
            pltpu.SemaphoreType.DMA,
            pltpu.SemaphoreType.DMA,
            pltpu.SemaphoreType.DMA,
            pltpu.SemaphoreType.DMA,
        ],
    )(_sc_gather)
    return k(token_ids_flat, wpairs)


def kernel(token_ids, weight):
    flat = token_ids.reshape(-1).astype(jnp.int32)
    w128 = jnp.concatenate([weight[0::2], weight[1::2]], axis=1)
    out = _embed(flat, w128)
    return out.reshape(token_ids.shape + (weight.shape[1],))


# speedup vs baseline: 8.5997x; 8.5997x over previous
import functools

import jax
import jax.numpy as jnp
from jax import lax
from jax.experimental import pallas as pl
from jax.experimental.pallas import tpu as pltpu
from jax.experimental.pallas import tpu_sc as plsc

_D = 64
_V = 1000000
_B = 4096 * 200

_info = plsc.get_sparse_core_info()
_NC, _NS = _info.num_cores, _info.num_subcores
_NW = _NC * _NS
_BPW = _B // _NW
_CHUNK = 160
_NCHUNK = _BPW // _CHUNK
_GW = _CHUNK // 2


def _sc_gather(idx_hbm, table_hbm, out_hbm,
               idxv0, idxv1, idx20, idx21, par0, par1,
               wide0, wide1, sel0, sel1,
               sem_idx, semg0, semg1, semo0, semo1):
    wid = lax.axis_index("s") * _NC + lax.axis_index("c")
    base = wid * _BPW
    idx_v = (idxv0, idxv1)
    idx2_v = (idx20, idx21)
    par_v = (par0, par1)
    wide_v = (wide0, wide1)
    sel_v = (sel0, sel1)
    sem_g = (semg0, semg1)
    sem_out = (semo0, semo1)

    def stage(i, s):
        off = pl.multiple_of(base + i * _CHUNK, _CHUNK)
        pltpu.make_async_copy(idx_hbm.at[pl.ds(off, _CHUNK)],
                              idx_v[s].at[pl.ds(0, _CHUNK)], sem_idx).wait()

        @pl.when(i + 1 < _NCHUNK)
        def _():
            offn = pl.multiple_of(base + (i + 1) * _CHUNK, _CHUNK)
            pltpu.async_copy(idx_hbm.at[pl.ds(offn, _CHUNK)],
                             idx_v[1 - s].at[pl.ds(0, _CHUNK)], sem_idx)

        for j in range(_CHUNK // 16):
            t = idx_v[s][pl.ds(j * 16, 16)]
            idx2_v[s][pl.ds(j * 16, 16)] = lax.shift_right_logical(t, 1)
            par_v[s][pl.ds(j * 16, 16)] = (t & 1) * 64
        for j in range(_CHUNK // _GW):
            pltpu.async_copy(
                table_hbm.at[idx2_v[s].at[pl.ds(j * _GW, _GW)]],
                wide_v[s].at[pl.ds(j * _GW, _GW)], sem_g[s])

    def complete(i, s, drain):
        off = pl.multiple_of(base + i * _CHUNK, _CHUNK)
        for j in range(_CHUNK // _GW):
            pltpu.make_async_copy(
                table_hbm.at[idx2_v[s].at[pl.ds(j * _GW, _GW)]],
                wide_v[s].at[pl.ds(j * _GW, _GW)], sem_g[s]).wait()
        if drain:
            @pl.when(i >= 2)
            def _():
                pltpu.make_async_copy(sel_v[s],
                                      out_hbm.at[pl.ds(off, _CHUNK)],
                                      sem_out[s]).wait()

        @plsc.parallel_loop(0, _CHUNK, unroll=8)
        def _(r):
            b = par_v[s][pl.ds(r, 16)][0]
            for j in range(4):
                sel_v[s][r, pl.ds(j * 16, 16)] = (
                    wide_v[s][r, pl.ds(b + j * 16, 16)])

        pltpu.async_copy(sel_v[s], out_hbm.at[pl.ds(off, _CHUNK)],
                         sem_out[s])

    pltpu.async_copy(idx_hbm.at[pl.ds(pl.multiple_of(base, _CHUNK), _CHUNK)],
                     idx_v[0].at[pl.ds(0, _CHUNK)], sem_idx)
    stage(0, 0)

    def pair(g, _):
        i = g * 2
        stage(i + 1, 1)
        complete(i, 0, True)

        @pl.when(i + 2 < _NCHUNK)
        def _():
            stage(i + 2, 0)
        complete(i + 1, 1, True)
        return ()

    lax.fori_loop(0, _NCHUNK // 2, pair, ())
    for s in range(2):
        off = pl.multiple_of(base + (_NCHUNK - 2 + s) * _CHUNK, _CHUNK)
        pltpu.make_async_copy(sel_v[s], out_hbm.at[pl.ds(off, _CHUNK)],
                              sem_out[s]).wait()


@jax.jit
def _embed(token_ids_flat, wpairs):
    mesh = plsc.VectorSubcoreMesh(core_axis_name="c", subcore_axis_name="s")
    k = functools.partial(
        pl.kernel,
        mesh=mesh,
        compiler_params=pltpu.CompilerParams(needs_layout_passes=False),
        out_type=jax.ShapeDtypeStruct((_B, _D), jnp.float32),
        scratch_types=[
            pltpu.VMEM((_CHUNK + 16,), jnp.int32),
            pltpu.VMEM((_CHUNK + 16,), jnp.int32),
            pltpu.VMEM((_CHUNK,), jnp.int32),
            pltpu.VMEM((_CHUNK,), jnp.int32),
            pltpu.VMEM((_CHUNK + 16,), jnp.int32),
            pltpu.VMEM((_CHUNK + 16,), jnp.int32),
            pltpu.VMEM((_CHUNK, 128), jnp.float32),
            pltpu.VMEM((_CHUNK, 128), jnp.float32),
            pltpu.VMEM((_CHUNK, _D), jnp.float32),
            pltpu.VMEM((_CHUNK, _D), jnp.float32),
            pltpu.SemaphoreType.DMA,
            pltpu.SemaphoreType.DMA,
            pltpu.SemaphoreType.DMA,
            pltpu.SemaphoreType.DMA,
            pltpu.SemaphoreType.DMA,
        ],
    )(_sc_gather)
    return k(token_ids_flat, wpairs)


def kernel(token_ids, weight):
    flat = token_ids.reshape(-1).astype(jnp.int32)
    w128 = weight.reshape(_V // 2, 128)
    out = _embed(flat, w128)
    return out.reshape(token_ids.shape + (weight.shape[1],))
